# baseline (speedup 1.0000x reference)
import functools

import jax
import jax.numpy as jnp
from jax import lax
from jax.experimental import pallas as pl
from jax.experimental.pallas import tpu as pltpu
from jax.experimental.pallas import tpu_sc as plsc

N = 10000
E = 320000
D = 128

NC = 2
NS = 16
NW = NC * NS

EDGES_PER_W = E // NW
CHUNK = 200
NCH_E = -(-EDGES_PER_W // CHUNK)
SELF_PER_W = 320
NCH_S = 2
NCHUNKS = NCH_E + NCH_S
TOT_PER_W = EDGES_PER_W + SELF_PER_W
NBUF = 4
PREF = 2


def _mlp_body(x_ref, w1_ref, b1_ref, w2_ref, b2_ref, o_ref):
    w1c = w1_ref[:, :D] + w1_ref[:, D:]
    h = lax.dot_general(x_ref[...], w1c, (((1,), (1,)), ((), ())),
                        preferred_element_type=jnp.float32)
    h = jnp.maximum(h + b1_ref[...], 0.0)
    o = lax.dot_general(h, w2_ref[...], (((1,), (1,)), ((), ())),
                        preferred_element_type=jnp.float32)
    o_ref[...] = jnp.maximum(o + b2_ref[...], 0.0)


def _node_mlp(x, w1, b1, w2, b2):
    blk = 2000
    grid = N // blk
    return pl.pallas_call(
        _mlp_body,
        grid=(grid,),
        in_specs=[
            pl.BlockSpec((blk, D), lambda i: (i, 0)),
            pl.BlockSpec((D, 2 * D), lambda i: (0, 0)),
            pl.BlockSpec((1, D), lambda i: (0, 0)),
            pl.BlockSpec((D, D), lambda i: (0, 0)),
            pl.BlockSpec((1, D), lambda i: (0, 0)),
        ],
        out_specs=pl.BlockSpec((blk, D), lambda i: (i, 0)),
        out_shape=jax.ShapeDtypeStruct((N, D), jnp.float32),
    )(x, w1, b1.reshape(1, D), w2, b2.reshape(1, D))


def _sc_gather(y, idx3):
    mesh = plsc.VectorSubcoreMesh(core_axis_name="c", subcore_axis_name="s")

    @functools.partial(
        pl.kernel,
        mesh=mesh,
        out_type=jax.ShapeDtypeStruct((N + E, D), jnp.float32),
        scratch_types=(
            [pltpu.VMEM((EDGES_PER_W,), jnp.int32),
             pltpu.VMEM((SELF_PER_W,), jnp.int32)]
            + [pltpu.VMEM((CHUNK, D), jnp.float32) for _ in range(NBUF)]
            + [pltpu.SemaphoreType.DMA for _ in range(2 * NBUF)]
        ),
    )
    def gather_kernel(y_hbm, idx_hbm, out_hbm, idx_v, sidx_v, *scratch):
        rows = scratch[:NBUF]
        gsem = scratch[NBUF:2 * NBUF]
        wsem = scratch[2 * NBUF:3 * NBUF]
        wid = lax.axis_index("s") * NC + lax.axis_index("c")

        pltpu.sync_copy(idx_hbm.at[wid], idx_v)

        dst0 = N + wid * EDGES_PER_W
        sbase = jnp.minimum(wid * SELF_PER_W, N - SELF_PER_W)

        lane = lax.broadcasted_iota(jnp.int32, (16,), 0)
        for k in range(SELF_PER_W // 16):
            sidx_v[pl.ds(16 * k, 16)] = sbase + 16 * k + lane

        def e_off(j):
            return jnp.minimum(j * CHUNK, EDGES_PER_W - CHUNK)

        def s_delta(j):
            return (j - NCH_E) * (SELF_PER_W - CHUNK)

        def out_at(j):
            dst = jnp.where(j < NCH_E, dst0 + e_off(j), sbase + s_delta(j))
            return out_hbm.at[pl.ds(dst, CHUNK)]

        def gather_op(j, b, wait):
            def run(idx_ref):
                cp = pltpu.make_async_copy(y_hbm.at[idx_ref], rows[b],
                                           gsem[b])
                cp.wait() if wait else cp.start()

            @pl.when(j < NCH_E)
            def _():
                run(idx_v.at[pl.ds(e_off(j), CHUNK)])
            @pl.when(j >= NCH_E)
            def _():
                run(sidx_v.at[pl.ds(s_delta(j), CHUNK)])

        for jj in range(PREF):
            gather_op(jnp.int32(jj), jj, wait=False)

        def step(j, carry):
            for b in range(NBUF):
                @pl.when(lax.rem(j, NBUF) == b)
                def _():
                    nb = (b + PREF) % NBUF
                    @pl.when(j + PREF >= NBUF)
                    def _():
                        pltpu.make_async_copy(
                            rows[nb], out_at(j + PREF - NBUF),
                            wsem[nb]).wait()
                    @pl.when(j + PREF < NCHUNKS)
                    def _():
                        gather_op(j + PREF, nb, wait=False)
                    gather_op(j, b, wait=True)
                    pltpu.async_copy(rows[b], out_at(j), wsem[b])
            return carry

        lax.fori_loop(0, NCHUNKS, step, 0)
        for j in range(NCHUNKS - (NBUF - PREF), NCHUNKS):
            pltpu.make_async_copy(rows[j % NBUF], out_at(j),
                                  wsem[j % NBUF]).wait()

    return gather_kernel(y, idx3)


def kernel(x, edge_index, W1, b1, W2, b2):
    y = _node_mlp(x, W1, b1, W2, b2)
    return _sc_gather(y, edge_index[0].reshape(NW, EDGES_PER_W))


# speedup vs baseline: 1.0291x; 1.0291x over previous
import functools

import jax
import jax.numpy as jnp
from jax import lax
from jax.experimental import pallas as pl
from jax.experimental.pallas import tpu as pltpu
from jax.experimental.pallas import tpu_sc as plsc

N = 10000
E = 320000
D = 128

NC = 2
NS = 16
NW = NC * NS

EDGES_PER_W = E // NW
CHUNK = 200
NCHUNKS = -(-EDGES_PER_W // CHUNK)
SELF_PER_W = 320
NBUF = 3
PREF = 1


def _mlp_body(x_ref, w1_ref, b1_ref, w2_ref, b2_ref, o_ref):
    w1c = w1_ref[:, :D] + w1_ref[:, D:]
    h = lax.dot_general(x_ref[...], w1c, (((1,), (1,)), ((), ())),
                        preferred_element_type=jnp.float32)
    h = jnp.maximum(h + b1_ref[...], 0.0)
    o = lax.dot_general(h, w2_ref[...], (((1,), (1,)), ((), ())),
                        preferred_element_type=jnp.float32)
    o_ref[...] = jnp.maximum(o + b2_ref[...], 0.0)


def _node_mlp(x, w1, b1, w2, b2):
    blk = 2000
    grid = N // blk
    return pl.pallas_call(
        _mlp_body,
        grid=(grid,),
        in_specs=[
            pl.BlockSpec((blk, D), lambda i: (i, 0)),
            pl.BlockSpec((D, 2 * D), lambda i: (0, 0)),
            pl.BlockSpec((1, D), lambda i: (0, 0)),
            pl.BlockSpec((D, D), lambda i: (0, 0)),
            pl.BlockSpec((1, D), lambda i: (0, 0)),
        ],
        out_specs=pl.BlockSpec((blk, D), lambda i: (i, 0)),
        out_shape=jax.ShapeDtypeStruct((N, D), jnp.float32),
    )(x, w1, b1.reshape(1, D), w2, b2.reshape(1, D))


def _sc_gather(y, idx3):
    mesh = plsc.VectorSubcoreMesh(core_axis_name="c", subcore_axis_name="s")

    @functools.partial(
        pl.kernel,
        mesh=mesh,
        out_type=jax.ShapeDtypeStruct((N + E, D), jnp.float32),
        scratch_types=(
            [pltpu.VMEM((EDGES_PER_W,), jnp.int32)]
            + [pltpu.VMEM((CHUNK, D), jnp.float32) for _ in range(NBUF)]
            + [pltpu.VMEM((SELF_PER_W, D), jnp.float32)]
            + [pltpu.SemaphoreType.DMA for _ in range(2 * NBUF + 1)]
        ),
    )
    def gather_kernel(y_hbm, idx_hbm, out_hbm, idx_v, *scratch):
        rows = scratch[:NBUF]
        self_v = scratch[NBUF]
        gsem = scratch[NBUF + 1:2 * NBUF + 1]
        wsem = scratch[2 * NBUF + 1:3 * NBUF + 1]
        ssem = scratch[3 * NBUF + 1]
        wid = lax.axis_index("s") * NC + lax.axis_index("c")

        base = jnp.minimum(wid * SELF_PER_W, N - SELF_PER_W)
        pltpu.async_copy(y_hbm.at[pl.ds(base, SELF_PER_W)], self_v, ssem)

        pltpu.sync_copy(idx_hbm.at[wid], idx_v)

        dst0 = N + wid * EDGES_PER_W

        def chunk_off(j):
            return jnp.minimum(j * CHUNK, EDGES_PER_W - CHUNK)

        def idx_at(j):
            return idx_v.at[pl.ds(chunk_off(j), CHUNK)]

        def out_at(j):
            return out_hbm.at[pl.ds(dst0 + chunk_off(j), CHUNK)]

        for jj in range(PREF):
            pltpu.async_copy(y_hbm.at[idx_at(jj)], rows[jj], gsem[jj])

        pltpu.make_async_copy(
            y_hbm.at[pl.ds(base, SELF_PER_W)], self_v, ssem).wait()
        pltpu.async_copy(self_v, out_hbm.at[pl.ds(base, SELF_PER_W)], ssem)

        def step(j, carry):
            for b in range(NBUF):
                @pl.when(lax.rem(j, NBUF) == b)
                def _():
                    nb = (b + PREF) % NBUF
                    @pl.when(j + PREF >= NBUF)
                    def _():
                        pltpu.make_async_copy(
                            rows[nb], out_at(j + PREF - NBUF),
                            wsem[nb]).wait()
                    @pl.when(j + PREF < NCHUNKS)
                    def _():
                        pltpu.async_copy(
                            y_hbm.at[idx_at(j + PREF)], rows[nb],
                            gsem[nb])
                    pltpu.make_async_copy(
                        y_hbm.at[idx_at(j)], rows[b], gsem[b]).wait()
                    pltpu.async_copy(rows[b], out_at(j), wsem[b])
            return carry

        lax.fori_loop(0, NCHUNKS, step, 0)
        for j in range(NCHUNKS - (NBUF - PREF), NCHUNKS):
            pltpu.make_async_copy(rows[j % NBUF], out_at(j),
                                  wsem[j % NBUF]).wait()
        pltpu.make_async_copy(
            self_v, out_hbm.at[pl.ds(base, SELF_PER_W)], ssem).wait()

    return gather_kernel(y, idx3)


def kernel(x, edge_index, W1, b1, W2, b2):
    y = _node_mlp(x, W1, b1, W2, b2)
    return _sc_gather(y, edge_index[0].reshape(NW, EDGES_PER_W))
